# baseline (speedup 1.0000x reference)
import functools

import jax
import jax.numpy as jnp
from jax import lax
from jax.experimental import pallas as pl
from jax.experimental.pallas import tpu as pltpu
from jax.experimental.pallas import tpu_sc as plsc

_B = 16
_S = 2048
_D = 256
_T = 16384

_NC = 2
_NS = 16
_NW = _NC * _NS
_P = 64
_NB = 6
_GP = (_B * _S) // _P
_PPB = _S // _P
_NPT = _GP // _NW
_NV = _D // 16


def _body(flat_hbm, cu_hbm, out_hbm, cu_v, idx0, idx1, idx2, idx3, idx4,
          idx5, zbuf, buf0, buf1, buf2, buf3, buf4, buf5, ssem0, ssem1,
          ssem2, ssem3, ssem4, ssem5, zsem, gsem0, gsem1, gsem2, gsem3,
          gsem4, gsem5):
    idxs = (idx0, idx1, idx2, idx3, idx4, idx5)
    bufs = (buf0, buf1, buf2, buf3, buf4, buf5)
    ssems = (ssem0, ssem1, ssem2, ssem3, ssem4, ssem5)
    gsems = (gsem0, gsem1, gsem2, gsem3, gsem4, gsem5)

    cid = lax.axis_index("c")
    sid = lax.axis_index("s")
    wid = sid * _NC + cid

    pltpu.sync_copy(cu_hbm, cu_v)
    lane = lax.iota(jnp.int32, 16)
    cu_lo = cu_v[pl.ds(0, 16)]

    zero16 = jnp.zeros((16,), jnp.float32)

    def zloop(r, carry):
        for c in range(_NV):
            zbuf[r, pl.ds(c * 16, 16)] = zero16
        return carry

    lax.fori_loop(0, _P, zloop, 0)

    def piece_info(g):
        b_g = g // _PPB
        s_off = (g % _PPB) * _P
        ivec = jnp.minimum(b_g + lane, _B - 1)
        gathered = cu_lo.at[ivec].get(mode="promise_in_bounds")
        start = gathered[0]
        end = jnp.where(b_g + 1 >= _B, _T, gathered[1])
        lim = jnp.minimum(end - start, _S)
        pv = jnp.clip(lim - s_off, 0, _P)
        return b_g, s_off, start + s_off, pv

    gs = [wid + _NW * j for j in range(_NPT)]
    infos = [piece_info(g) for g in gs]
    pvs = [inf[3] for inf in infos]

    def dst(b_g, s_off):
        return out_hbm.at[b_g, pl.ds(s_off, _P), :]

    for j in range(_NPT):
        b_g, s_off, _, _ = infos[j]

        @pl.when(pvs[j] == 0)
        def _zstore(b_g=b_g, s_off=s_off):
            pltpu.make_async_copy(zbuf, dst(b_g, s_off), zsem).start()

    def fill_idx(i_ref, src):
        for k in range(_P // 16):
            i_ref[pl.ds(k * 16, 16)] = jnp.minimum(src + k * 16 + lane,
                                                   _T - 1)

    b0, so0, _, _ = infos[0]

    def gather(slot):
        return pltpu.make_async_copy(flat_hbm.at[idxs[slot]], bufs[slot],
                                     gsems[slot])

    sfired = [jnp.int32(0)] * _NB

    for d in range(_NB - 1):
        @pl.when(pvs[d] > 0)
        def _pro(d=d):
            fill_idx(idxs[d], infos[d][2])
            gather(d).start()

    for j in range(_NPT):
        slot = j % _NB
        b_g, s_off, _, pv = infos[j]

        @pl.when(pv > 0)
        def _gw(slot=slot):
            gather(slot).wait()

        jn = j + _NB - 1
        if jn < _NPT:
            nslot = jn % _NB

            @pl.when(pvs[jn] > 0)
            def _pre(jn=jn, nslot=nslot, sf=sfired[nslot]):
                @pl.when(sf > 0)
                def _fs(nslot=nslot):
                    pltpu.make_async_copy(bufs[nslot], dst(b0, so0),
                                          ssems[nslot]).wait()

                fill_idx(idxs[nslot], infos[jn][2])
                gather(nslot).start()

            sfired[nslot] = jnp.where(pvs[jn] > 0, jnp.int32(0),
                                      sfired[nslot])

        @pl.when(pv > 0)
        def _st(slot=slot, b_g=b_g, s_off=s_off, pv=pv):
            @pl.when(pv < _P)
            def _tail():
                def tloop(r, c2):
                    for c in range(_NV):
                        bufs[slot][r, pl.ds(c * 16, 16)] = zero16
                    return c2

                lax.fori_loop(pv, _P, tloop, 0)

            pltpu.make_async_copy(bufs[slot], dst(b_g, s_off),
                                  ssems[slot]).start()

        sfired[slot] = jnp.where(pv > 0, jnp.int32(1), sfired[slot])

    for slot in range(_NB):
        @pl.when(sfired[slot] > 0)
        def _ds(slot=slot):
            pltpu.make_async_copy(bufs[slot], dst(b0, so0),
                                  ssems[slot]).wait()

    nz = jnp.int32(0)
    for j in range(_NPT):
        nz = nz + jnp.where(pvs[j] == 0, 1, 0)

    def zdrain(i, carry):
        pltpu.make_async_copy(zbuf, dst(b0, so0), zsem).wait()
        return carry

    lax.fori_loop(0, nz, zdrain, 0)


_sc_kernel = functools.partial(
    pl.kernel,
    out_type=jax.ShapeDtypeStruct((_B, _S, _D), jnp.float32),
    mesh=plsc.VectorSubcoreMesh(core_axis_name="c", subcore_axis_name="s"),
    scratch_types=[
        pltpu.VMEM((_B + 1,), jnp.int32),
        pltpu.VMEM((_P,), jnp.int32),
        pltpu.VMEM((_P,), jnp.int32),
        pltpu.VMEM((_P,), jnp.int32),
        pltpu.VMEM((_P,), jnp.int32),
        pltpu.VMEM((_P,), jnp.int32),
        pltpu.VMEM((_P,), jnp.int32),
        pltpu.VMEM((_P, _D), jnp.float32),
        pltpu.VMEM((_P, _D), jnp.float32),
        pltpu.VMEM((_P, _D), jnp.float32),
        pltpu.VMEM((_P, _D), jnp.float32),
        pltpu.VMEM((_P, _D), jnp.float32),
        pltpu.VMEM((_P, _D), jnp.float32),
        pltpu.VMEM((_P, _D), jnp.float32),
        pltpu.SemaphoreType.DMA,
        pltpu.SemaphoreType.DMA,
        pltpu.SemaphoreType.DMA,
        pltpu.SemaphoreType.DMA,
        pltpu.SemaphoreType.DMA,
        pltpu.SemaphoreType.DMA,
        pltpu.SemaphoreType.DMA,
        pltpu.SemaphoreType.DMA,
        pltpu.SemaphoreType.DMA,
        pltpu.SemaphoreType.DMA,
        pltpu.SemaphoreType.DMA,
        pltpu.SemaphoreType.DMA,
        pltpu.SemaphoreType.DMA,
    ],
)(_body)


@jax.jit
def kernel(flat_values, cu_seqlens):
    return _sc_kernel(flat_values, cu_seqlens)


# speedup vs baseline: 1.0156x; 1.0156x over previous
import functools

import jax
import jax.numpy as jnp
from jax import lax
from jax.experimental import pallas as pl
from jax.experimental.pallas import tpu as pltpu
from jax.experimental.pallas import tpu_sc as plsc

_B = 16
_S = 2048
_D = 256
_T = 16384

_NC = 2
_NS = 16
_NW = _NC * _NS
_P = 64
_NB = 4
_GP = (_B * _S) // _P
_PPB = _S // _P
_NPT = _GP // _NW
_NV = _D // 16


def _body(flat_hbm, cu_hbm, out_hbm, cu_v, idx0, idx1, idx2, idx3, zbuf,
          buf0, buf1, buf2, buf3, ssem0, ssem1, ssem2, ssem3, zsem,
          gsem0, gsem1, gsem2, gsem3):
    idxs = (idx0, idx1, idx2, idx3)
    bufs = (buf0, buf1, buf2, buf3)
    ssems = (ssem0, ssem1, ssem2, ssem3)
    gsems = (gsem0, gsem1, gsem2, gsem3)

    cid = lax.axis_index("c")
    sid = lax.axis_index("s")
    wid = sid * _NC + cid

    pltpu.sync_copy(cu_hbm, cu_v)
    lane = lax.iota(jnp.int32, 16)
    cu_lo = cu_v[pl.ds(0, 16)]

    zero16 = jnp.zeros((16,), jnp.float32)

    def zloop(r, carry):
        for c in range(_NV):
            zbuf[r, pl.ds(c * 16, 16)] = zero16
        return carry

    lax.fori_loop(0, _P, zloop, 0)

    def piece_info(g):
        b_g = g // _PPB
        s_off = (g % _PPB) * _P
        ivec = jnp.minimum(b_g + lane, _B - 1)
        gathered = cu_lo.at[ivec].get(mode="promise_in_bounds")
        start = gathered[0]
        end = jnp.where(b_g + 1 >= _B, _T, gathered[1])
        lim = jnp.minimum(end - start, _S)
        pv = jnp.clip(lim - s_off, 0, _P)
        return b_g, s_off, start + s_off, pv

    gs = [wid + _NW * j for j in range(_NPT)]
    infos = [piece_info(g) for g in gs]
    pvs = [inf[3] for inf in infos]

    def dst(b_g, s_off):
        return out_hbm.at[b_g, pl.ds(s_off, _P), :]

    for j in range(_NPT):
        b_g, s_off, _, _ = infos[j]

        @pl.when(pvs[j] == 0)
        def _zstore(b_g=b_g, s_off=s_off):
            pltpu.make_async_copy(zbuf, dst(b_g, s_off), zsem).start()

    def fill_idx(i_ref, src):
        for k in range(_P // 16):
            i_ref[pl.ds(k * 16, 16)] = jnp.minimum(src + k * 16 + lane,
                                                   _T - 1)

    b0, so0, _, _ = infos[0]

    def gather(slot):
        return pltpu.make_async_copy(flat_hbm.at[idxs[slot]], bufs[slot],
                                     gsems[slot])

    sfired = [jnp.int32(0)] * _NB

    for d in range(_NB - 1):
        @pl.when(pvs[d] > 0)
        def _pro(d=d):
            fill_idx(idxs[d], infos[d][2])
            gather(d).start()

    for j in range(_NPT):
        slot = j % _NB
        b_g, s_off, _, pv = infos[j]

        @pl.when(pv > 0)
        def _gw(slot=slot):
            gather(slot).wait()

        jn = j + _NB - 1
        if jn < _NPT:
            nslot = jn % _NB

            @pl.when(pvs[jn] > 0)
            def _pre(jn=jn, nslot=nslot, sf=sfired[nslot]):
                @pl.when(sf > 0)
                def _fs(nslot=nslot):
                    pltpu.make_async_copy(bufs[nslot], dst(b0, so0),
                                          ssems[nslot]).wait()

                fill_idx(idxs[nslot], infos[jn][2])
                gather(nslot).start()

            sfired[nslot] = jnp.where(pvs[jn] > 0, jnp.int32(0),
                                      sfired[nslot])

        @pl.when(pv > 0)
        def _st(slot=slot, b_g=b_g, s_off=s_off, pv=pv):
            @pl.when(pv < _P)
            def _tail():
                def tloop(r, c2):
                    for c in range(_NV):
                        bufs[slot][r, pl.ds(c * 16, 16)] = zero16
                    return c2

                lax.fori_loop(pv, _P, tloop, 0)

            pltpu.make_async_copy(bufs[slot], dst(b_g, s_off),
                                  ssems[slot]).start()

        sfired[slot] = jnp.where(pv > 0, jnp.int32(1), sfired[slot])

    for slot in range(_NB):
        @pl.when(sfired[slot] > 0)
        def _ds(slot=slot):
            pltpu.make_async_copy(bufs[slot], dst(b0, so0),
                                  ssems[slot]).wait()

    nz = jnp.int32(0)
    for j in range(_NPT):
        nz = nz + jnp.where(pvs[j] == 0, 1, 0)

    def zdrain(i, carry):
        pltpu.make_async_copy(zbuf, dst(b0, so0), zsem).wait()
        return carry

    lax.fori_loop(0, nz, zdrain, 0)


_sc_kernel = functools.partial(
    pl.kernel,
    out_type=jax.ShapeDtypeStruct((_B, _S, _D), jnp.float32),
    mesh=plsc.VectorSubcoreMesh(core_axis_name="c", subcore_axis_name="s"),
    scratch_types=[
        pltpu.VMEM((_B + 1,), jnp.int32),
        pltpu.VMEM((_P,), jnp.int32),
        pltpu.VMEM((_P,), jnp.int32),
        pltpu.VMEM((_P,), jnp.int32),
        pltpu.VMEM((_P,), jnp.int32),
        pltpu.VMEM((_P, _D), jnp.float32),
        pltpu.VMEM((_P, _D), jnp.float32),
        pltpu.VMEM((_P, _D), jnp.float32),
        pltpu.VMEM((_P, _D), jnp.float32),
        pltpu.VMEM((_P, _D), jnp.float32),
        pltpu.SemaphoreType.DMA,
        pltpu.SemaphoreType.DMA,
        pltpu.SemaphoreType.DMA,
        pltpu.SemaphoreType.DMA,
        pltpu.SemaphoreType.DMA,
        pltpu.SemaphoreType.DMA,
        pltpu.SemaphoreType.DMA,
        pltpu.SemaphoreType.DMA,
        pltpu.SemaphoreType.DMA,
    ],
)(_body)


@jax.jit
def kernel(flat_values, cu_seqlens):
    return _sc_kernel(flat_values, cu_seqlens)


# speedup vs baseline: 1.0178x; 1.0022x over previous
import functools

import jax
import jax.numpy as jnp
from jax import lax
from jax.experimental import pallas as pl
from jax.experimental.pallas import tpu as pltpu
from jax.experimental.pallas import tpu_sc as plsc

_B = 16
_S = 2048
_D = 256
_T = 16384

_NC = 2
_NS = 16
_NW = _NC * _NS
_P = 64
_NB = 4
_GP = (_B * _S) // _P
_PPB = _S // _P
_NPT = _GP // _NW
_NV = _D // 16


def _body(flat_hbm, cu_hbm, out_hbm, cu_v, idx0, idx1, idx2, idx3, zbuf,
          buf0, buf1, buf2, buf3, ssem0, ssem1, ssem2, ssem3, zsem,
          gsem0, gsem1, gsem2, gsem3):
    idxs = (idx0, idx1, idx2, idx3)
    bufs = (buf0, buf1, buf2, buf3)
    ssems = (ssem0, ssem1, ssem2, ssem3)
    gsems = (gsem0, gsem1, gsem2, gsem3)

    cid = lax.axis_index("c")
    sid = lax.axis_index("s")
    wid = sid * _NC + cid

    pltpu.sync_copy(cu_hbm, cu_v)
    lane = lax.iota(jnp.int32, 16)
    cu_lo = cu_v[pl.ds(0, 16)]

    zero16 = jnp.zeros((16,), jnp.float32)

    def zloop(r, carry):
        for c in range(_NV):
            zbuf[r, pl.ds(c * 16, 16)] = zero16
        return carry

    lax.fori_loop(0, _P, zloop, 0)

    s_off = wid * _P

    def piece_info(j):
        start = cu_lo[j]
        end = cu_lo[j + 1] if j + 1 < _B else _T
        lim = jnp.minimum(end - start, _S)
        pv = jnp.clip(lim - s_off, 0, _P)
        return j, s_off, start + s_off, pv

    infos = [piece_info(j) for j in range(_NPT)]
    pvs = [inf[3] for inf in infos]

    def dst(b_g, s_off):
        return out_hbm.at[b_g, pl.ds(s_off, _P), :]

    for j in range(_NPT):
        b_g, s_off, _, _ = infos[j]

        @pl.when(pvs[j] == 0)
        def _zstore(b_g=b_g, s_off=s_off):
            pltpu.make_async_copy(zbuf, dst(b_g, s_off), zsem).start()

    def fill_idx(i_ref, src):
        for k in range(_P // 16):
            i_ref[pl.ds(k * 16, 16)] = jnp.minimum(src + k * 16 + lane,
                                                   _T - 1)

    b0, so0, _, _ = infos[0]

    def gather(slot):
        return pltpu.make_async_copy(flat_hbm.at[idxs[slot]], bufs[slot],
                                     gsems[slot])

    sfired = [jnp.int32(0)] * _NB

    for d in range(_NB - 1):
        @pl.when(pvs[d] > 0)
        def _pro(d=d):
            fill_idx(idxs[d], infos[d][2])
            gather(d).start()

    for j in range(_NPT):
        slot = j % _NB
        b_g, s_off, _, pv = infos[j]

        @pl.when(pv > 0)
        def _gw(slot=slot):
            gather(slot).wait()

        jn = j + _NB - 1
        if jn < _NPT:
            nslot = jn % _NB

            @pl.when(pvs[jn] > 0)
            def _pre(jn=jn, nslot=nslot, sf=sfired[nslot]):
                @pl.when(sf > 0)
                def _fs(nslot=nslot):
                    pltpu.make_async_copy(bufs[nslot], dst(b0, so0),
                                          ssems[nslot]).wait()

                fill_idx(idxs[nslot], infos[jn][2])
                gather(nslot).start()

            sfired[nslot] = jnp.where(pvs[jn] > 0, jnp.int32(0),
                                      sfired[nslot])

        @pl.when(pv > 0)
        def _st(slot=slot, b_g=b_g, s_off=s_off, pv=pv):
            @pl.when(pv < _P)
            def _tail():
                def tloop(r, c2):
                    for c in range(_NV):
                        bufs[slot][r, pl.ds(c * 16, 16)] = zero16
                    return c2

                lax.fori_loop(pv, _P, tloop, 0)

            pltpu.make_async_copy(bufs[slot], dst(b_g, s_off),
                                  ssems[slot]).start()

        sfired[slot] = jnp.where(pv > 0, jnp.int32(1), sfired[slot])

    for slot in range(_NB):
        @pl.when(sfired[slot] > 0)
        def _ds(slot=slot):
            pltpu.make_async_copy(bufs[slot], dst(b0, so0),
                                  ssems[slot]).wait()

    nz = jnp.int32(0)
    for j in range(_NPT):
        nz = nz + jnp.where(pvs[j] == 0, 1, 0)

    def zdrain(i, carry):
        pltpu.make_async_copy(zbuf, dst(b0, so0), zsem).wait()
        return carry

    lax.fori_loop(0, nz, zdrain, 0)


_sc_kernel = functools.partial(
    pl.kernel,
    out_type=jax.ShapeDtypeStruct((_B, _S, _D), jnp.float32),
    mesh=plsc.VectorSubcoreMesh(core_axis_name="c", subcore_axis_name="s"),
    scratch_types=[
        pltpu.VMEM((_B + 1,), jnp.int32),
        pltpu.VMEM((_P,), jnp.int32),
        pltpu.VMEM((_P,), jnp.int32),
        pltpu.VMEM((_P,), jnp.int32),
        pltpu.VMEM((_P,), jnp.int32),
        pltpu.VMEM((_P, _D), jnp.float32),
        pltpu.VMEM((_P, _D), jnp.float32),
        pltpu.VMEM((_P, _D), jnp.float32),
        pltpu.VMEM((_P, _D), jnp.float32),
        pltpu.VMEM((_P, _D), jnp.float32),
        pltpu.SemaphoreType.DMA,
        pltpu.SemaphoreType.DMA,
        pltpu.SemaphoreType.DMA,
        pltpu.SemaphoreType.DMA,
        pltpu.SemaphoreType.DMA,
        pltpu.SemaphoreType.DMA,
        pltpu.SemaphoreType.DMA,
        pltpu.SemaphoreType.DMA,
        pltpu.SemaphoreType.DMA,
    ],
)(_body)


@jax.jit
def kernel(flat_values, cu_seqlens):
    return _sc_kernel(flat_values, cu_seqlens)
